# baseline (speedup 1.0000x reference)
import functools

import jax
import jax.numpy as jnp
from jax import lax
from jax.experimental import pallas as pl
from jax.experimental.pallas import tpu as pltpu
from jax.experimental.pallas import tpu_sc as plsc

_PRIMES = [100003, 100019, 100043, 100049, 100057, 100069, 100103, 100109]
_ED = 16
_LANES = 16
_NC = 2
_NS = 16
_NW = _NC * _NS
_RING = 4


def _head_offsets():
    offs = [0]
    for p in _PRIMES[:-1]:
        offs.append(offs[-1] + p)
    return offs


def _make_kernel(b, s, h):
    bpt = b // _NW
    units = s * h
    offs = _head_offsets()
    mesh = plsc.VectorSubcoreMesh(core_axis_name="c", subcore_axis_name="s")

    @functools.partial(
        pl.kernel,
        mesh=mesh,
        compiler_params=pltpu.CompilerParams(
            use_tc_tiling_on_sc=False, needs_layout_passes=False
        ),
        out_type=jax.ShapeDtypeStruct(
            (s, h, _ED // 8, _NW, 8, bpt), jnp.float32
        ),
        scratch_types=[
            pltpu.VMEM((s, h, bpt), jnp.int32),
        ]
        + [pltpu.VMEM((bpt, _ED), jnp.float32) for _ in range(_RING)]
        + [pltpu.VMEM((_ED // 8, 8, bpt), jnp.float32) for _ in range(_RING)]
        + [pltpu.SemaphoreType.DMA for _ in range(2 * _RING)],
    )
    def k(w_hbm, idx_hbm, out_hbm, idx_v, *rest):
        rows = rest[:_RING]
        trans = rest[_RING : 2 * _RING]
        gsem = rest[2 * _RING : 3 * _RING]
        osem = rest[3 * _RING : 4 * _RING]
        t = lax.axis_index("s") * _NC + lax.axis_index("c")
        pltpu.sync_copy(idx_hbm.at[:, t], idx_v)

        def preadd(si, c):
            for hi in range(h):
                o = jnp.int32(offs[hi])
                for j in range(bpt // _LANES):
                    sl = (si, hi, pl.ds(j * _LANES, _LANES))
                    idx_v[sl] = idx_v[sl] + o
            return c

        lax.fori_loop(0, s, preadd, 0)

        iota16 = lax.iota(jnp.int32, _LANES)
        rvecs = [iota16 + jnp.int32(j * _LANES) for j in range(bpt // _LANES)]
        dvecs = [jnp.full((_LANES,), d, jnp.int32) for d in range(_ED)]

        def fire_gather(u, slot):
            si, hi = u // h, u % h
            pltpu.async_copy(w_hbm.at[idx_v.at[si, hi]], rows[slot], gsem[slot])

        def wait_gather(u, slot):
            si, hi = u // h, u % h
            pltpu.make_async_copy(
                w_hbm.at[idx_v.at[si, hi]], rows[slot], gsem[slot]
            ).wait()

        def out_ref(u):
            si, hi = u // h, u % h
            return out_hbm.at[si, hi, :, t]

        def transpose(slot):
            for j in range(bpt // _LANES):
                vs = [
                    plsc.load_gather(rows[slot], [rvecs[j], dvecs[d]])
                    for d in range(_ED)
                ]
                for d in range(_ED):
                    trans[slot][d // 8, d % 8, pl.ds(j * _LANES, _LANES)] = vs[d]

        def step(u, slot, first):
            nxt = u + _RING - 1
            if isinstance(nxt, int):
                if nxt < units:
                    fire_gather(nxt, (slot + _RING - 1) % _RING)
            else:

                @pl.when(nxt < units)
                def _():
                    fire_gather(nxt, (slot + _RING - 1) % _RING)

            wait_gather(u, slot)
            if not first:
                pltpu.make_async_copy(trans[slot], out_ref(u), osem[slot]).wait()
            transpose(slot)
            pltpu.async_copy(trans[slot], out_ref(u), osem[slot])

        for uu in range(_RING - 1):
            fire_gather(uu, uu)
        for kk in range(_RING):
            step(kk, kk, first=True)

        def body(g, c):
            for kk in range(_RING):
                step(g * _RING + kk, kk, first=False)
            return c

        lax.fori_loop(1, units // _RING, body, 0)

        for kk in range(_RING):
            u = units - _RING + kk
            pltpu.make_async_copy(trans[kk], out_ref(u), osem[kk]).wait()

    return k


def kernel(hash_indices, weight):
    b, s, h = hash_indices.shape
    bpt = b // _NW
    idx_t = (
        jnp.transpose(hash_indices.astype(jnp.int32), (1, 2, 0))
        .reshape(s, h, _NW, bpt)
        .transpose(0, 2, 1, 3)
    )
    sc = s // 2
    k2 = _make_kernel(b, sc, h)
    outs = [k2(weight, idx_t[:sc]), k2(weight, idx_t[sc:])]
    halves = [
        jnp.transpose(o, (3, 5, 0, 1, 2, 4)).reshape(b, sc, h, _ED)
        for o in outs
    ]
    return jnp.concatenate(halves, axis=1)


# speedup vs baseline: 1.1906x; 1.1906x over previous
import functools

import jax
import jax.numpy as jnp
from jax import lax
from jax.experimental import pallas as pl
from jax.experimental.pallas import tpu as pltpu
from jax.experimental.pallas import tpu_sc as plsc

_PRIMES = [100003, 100019, 100043, 100049, 100057, 100069, 100103, 100109]
_ED = 16
_LANES = 16
_NC = 2
_NS = 16
_NW = _NC * _NS
_RING = 5


def _head_offsets():
    offs = [0]
    for p in _PRIMES[:-1]:
        offs.append(offs[-1] + p)
    return offs


def _make_kernel(b, s, h):
    bpt = b // _NW
    units = s * h
    offs = _head_offsets()
    mesh = plsc.VectorSubcoreMesh(core_axis_name="c", subcore_axis_name="s")

    @functools.partial(
        pl.kernel,
        mesh=mesh,
        compiler_params=pltpu.CompilerParams(
            use_tc_tiling_on_sc=False, needs_layout_passes=False
        ),
        out_type=jax.ShapeDtypeStruct(
            (s, h, _ED // 8, _NW, 8, bpt), jnp.float32
        ),
        scratch_types=[
            pltpu.VMEM((s, h, bpt), jnp.int32),
        ]
        + [pltpu.VMEM((bpt, _ED), jnp.float32) for _ in range(_RING)]
        + [pltpu.VMEM((_ED // 8, 8, bpt), jnp.float32) for _ in range(_RING)]
        + [pltpu.SemaphoreType.DMA for _ in range(2 * _RING)],
    )
    def k(w_hbm, idx_hbm, out_hbm, idx_v, *rest):
        rows = rest[:_RING]
        trans = rest[_RING : 2 * _RING]
        gsem = rest[2 * _RING : 3 * _RING]
        osem = rest[3 * _RING : 4 * _RING]
        t = lax.axis_index("s") * _NC + lax.axis_index("c")
        pltpu.sync_copy(idx_hbm.at[:, t], idx_v)

        def preadd(si, c):
            for hi in range(h):
                o = jnp.int32(offs[hi])
                for j in range(bpt // _LANES):
                    sl = (si, hi, pl.ds(j * _LANES, _LANES))
                    idx_v[sl] = idx_v[sl] + o
            return c

        lax.fori_loop(0, s, preadd, 0)

        iota16 = lax.iota(jnp.int32, _LANES)
        rvecs = [iota16 + jnp.int32(j * _LANES) for j in range(bpt // _LANES)]
        dvecs = [jnp.full((_LANES,), d, jnp.int32) for d in range(_ED)]

        def fire_gather(u, slot):
            si, hi = u // h, u % h
            pltpu.async_copy(w_hbm.at[idx_v.at[si, hi]], rows[slot], gsem[slot])

        def wait_gather(u, slot):
            si, hi = u // h, u % h
            pltpu.make_async_copy(
                w_hbm.at[idx_v.at[si, hi]], rows[slot], gsem[slot]
            ).wait()

        def out_ref(u):
            si, hi = u // h, u % h
            return out_hbm.at[si, hi, :, t]

        def transpose(slot):
            for j in range(bpt // _LANES):
                vs = [
                    plsc.load_gather(rows[slot], [rvecs[j], dvecs[d]])
                    for d in range(_ED)
                ]
                for d in range(_ED):
                    trans[slot][d // 8, d % 8, pl.ds(j * _LANES, _LANES)] = vs[d]

        def step(u, slot, first):
            nxt = u + _RING - 1
            if isinstance(nxt, int):
                if nxt < units:
                    fire_gather(nxt, (slot + _RING - 1) % _RING)
            else:

                @pl.when(nxt < units)
                def _():
                    fire_gather(nxt, (slot + _RING - 1) % _RING)

            wait_gather(u, slot)
            if not first:
                pltpu.make_async_copy(trans[slot], out_ref(u), osem[slot]).wait()
            transpose(slot)
            pltpu.async_copy(trans[slot], out_ref(u), osem[slot])

        for uu in range(_RING - 1):
            fire_gather(uu, uu)
        for kk in range(_RING):
            step(kk, kk, first=True)

        def body(g, c):
            for kk in range(_RING):
                step(g * _RING + kk, kk, first=False)
            return c

        lax.fori_loop(1, units // _RING, body, 0)

        for kk in range(_RING):
            u = units - _RING + kk
            pltpu.make_async_copy(trans[kk], out_ref(u), osem[kk]).wait()

    return k


def kernel(hash_indices, weight):
    b, s, h = hash_indices.shape
    bpt = b // _NW
    idx_t = (
        jnp.transpose(hash_indices.astype(jnp.int32), (1, 2, 0))
        .reshape(s, h, _NW, bpt)
        .transpose(0, 2, 1, 3)
    )
    out6 = _make_kernel(b, s, h)(weight, idx_t)
    return jnp.transpose(out6, (3, 5, 0, 1, 2, 4)).reshape(b, s, h, _ED)
